# baseline (speedup 1.0000x reference)
import jax
import jax.numpy as jnp
from jax.experimental import pallas as pl
from jax.experimental.pallas import tpu as pltpu

K = 32
EPS = 1e-6
BLK = 512


def _stats_kernel(z_ref, x_ref, r_ref, ns_ref, mus_ref, covs_ref,
                  *, nblocks, s, d):
    step = pl.program_id(0)
    g = K * s

    @pl.when(step == 0)
    def _init():
        ns_ref[...] = jnp.zeros_like(ns_ref)
        mus_ref[...] = jnp.zeros_like(mus_ref)
        covs_ref[...] = jnp.zeros_like(covs_ref)

    x = x_ref[...]
    r = r_ref[...]
    zc = z_ref[...]
    zmin = zc[0, 0]
    zmax = zc[BLK - 1, 0]

    lane = jax.lax.broadcasted_iota(jnp.int32, (BLK, g), 1)
    rcols = jnp.broadcast_to(r[:, 0:1], (BLK, g))
    for j in range(1, s):
        rcols = jnp.where(lane % s == j,
                          jnp.broadcast_to(r[:, j:j + 1], (BLK, g)), rcols)
    w_dense = jnp.where(zc == lane // s, rcols, 0.0)
    xa = jnp.concatenate(
        [x, jnp.ones((BLK, 1), jnp.float32)], axis=1)
    y2 = jax.lax.dot_general(
        w_dense, xa, (((0,), (0,)), ((), ())),
        preferred_element_type=jnp.float32,
        precision=jax.lax.Precision.HIGHEST)
    mus_ref[...] += y2[:, 0:d]
    ns_ref[...] += y2[:, d:d + 1]

    a = jnp.concatenate([x * r[:, j:j + 1] for j in range(s)],
                        axis=1)

    def seg_body(i, carry):
        for u in range(2):
            k = zmin + i * 2 + u
            xm = jnp.where(zc == k, x, 0.0)
            y = jax.lax.dot_general(
                a, xm, (((0,), (0,)), ((), ())),
                preferred_element_type=jnp.float32,
                precision=jax.lax.Precision.HIGHEST)
            kc = jnp.minimum(k, K - 1)
            for j in range(s):
                covs_ref[pl.ds(kc * s + j, 1)] += (
                    y[j * d:(j + 1) * d, :][None])
        return carry

    jax.lax.fori_loop(0, (zmax - zmin + 2) // 2, seg_body, 0)

    @pl.when(step == nblocks - 1)
    def _epilogue():
        denom = ns_ref[...] + EPS
        mus = mus_ref[...] / denom
        mus_ref[...] = mus
        covs_ref[...] = (covs_ref[...] / denom[:, :, None]
                         - mus[:, :, None] * mus[:, None, :])


def kernel(X, z, r):
    n, d = X.shape
    s = r.shape[1]
    g = K * s
    nblocks = n // BLK
    z2 = z.astype(jnp.int32).reshape(n, 1)

    ns, mus, covs = pl.pallas_call(
        lambda *refs: _stats_kernel(*refs, nblocks=nblocks, s=s, d=d),
        grid=(nblocks,),
        in_specs=[
            pl.BlockSpec((BLK, 1), lambda i: (i, 0)),
            pl.BlockSpec((BLK, d), lambda i: (i, 0)),
            pl.BlockSpec((BLK, s), lambda i: (i, 0)),
        ],
        out_specs=[
            pl.BlockSpec((g, 1), lambda i: (0, 0)),
            pl.BlockSpec((g, d), lambda i: (0, 0)),
            pl.BlockSpec((g, d, d), lambda i: (0, 0, 0)),
        ],
        out_shape=[
            jax.ShapeDtypeStruct((g, 1), jnp.float32),
            jax.ShapeDtypeStruct((g, d), jnp.float32),
            jax.ShapeDtypeStruct((g, d, d), jnp.float32),
        ],
        compiler_params=pltpu.CompilerParams(
            dimension_semantics=("arbitrary",)),
    )(z2, X, r)

    return ns.reshape(g), mus, covs.reshape(g, d, d)


# speedup vs baseline: 1.2269x; 1.2269x over previous
import jax
import jax.numpy as jnp
from jax.experimental import pallas as pl
from jax.experimental.pallas import tpu as pltpu

K = 32
EPS = 1e-6
BLK = 512


def _stats_kernel(z_ref, x_ref, r_ref, ns_ref, mus_ref, covs_ref,
                  *, nblocks, s, d):
    step = pl.program_id(0)
    g = K * s

    @pl.when(step == 0)
    def _init():
        ns_ref[...] = jnp.zeros_like(ns_ref)
        mus_ref[...] = jnp.zeros_like(mus_ref)
        covs_ref[...] = jnp.zeros_like(covs_ref)

    x = x_ref[...]
    r = r_ref[...]
    zc = z_ref[...]
    zmin = zc[0, 0]
    zmax = zc[BLK - 1, 0]

    lane = jax.lax.broadcasted_iota(jnp.int32, (BLK, g), 1)
    rcols = jnp.broadcast_to(r[:, 0:1], (BLK, g))
    for j in range(1, s):
        rcols = jnp.where(lane % s == j,
                          jnp.broadcast_to(r[:, j:j + 1], (BLK, g)), rcols)
    w_dense = jnp.where(zc == lane // s, rcols, 0.0)
    xa = jnp.concatenate(
        [x, jnp.ones((BLK, 1), jnp.float32)], axis=1)
    y2 = jax.lax.dot_general(
        w_dense, xa, (((0,), (0,)), ((), ())),
        preferred_element_type=jnp.float32)
    mus_ref[...] += y2[:, 0:d]
    ns_ref[...] += y2[:, d:d + 1]

    a = jnp.concatenate([x * r[:, j:j + 1] for j in range(s)],
                        axis=1)

    def seg_body(i, carry):
        for u in range(2):
            k = zmin + i * 2 + u
            xm = jnp.where(zc == k, x, 0.0)
            y = jax.lax.dot_general(
                a, xm, (((0,), (0,)), ((), ())),
                preferred_element_type=jnp.float32)
            kc = jnp.minimum(k, K - 1)
            for j in range(s):
                covs_ref[pl.ds(kc * s + j, 1)] += (
                    y[j * d:(j + 1) * d, :][None])
        return carry

    jax.lax.fori_loop(0, (zmax - zmin + 2) // 2, seg_body, 0)

    @pl.when(step == nblocks - 1)
    def _epilogue():
        denom = ns_ref[...] + EPS
        mus = mus_ref[...] / denom
        mus_ref[...] = mus
        covs_ref[...] = (covs_ref[...] / denom[:, :, None]
                         - mus[:, :, None] * mus[:, None, :])


def kernel(X, z, r):
    n, d = X.shape
    s = r.shape[1]
    g = K * s
    nblocks = n // BLK
    z2 = z.astype(jnp.int32).reshape(n, 1)

    ns, mus, covs = pl.pallas_call(
        lambda *refs: _stats_kernel(*refs, nblocks=nblocks, s=s, d=d),
        grid=(nblocks,),
        in_specs=[
            pl.BlockSpec((BLK, 1), lambda i: (i, 0)),
            pl.BlockSpec((BLK, d), lambda i: (i, 0)),
            pl.BlockSpec((BLK, s), lambda i: (i, 0)),
        ],
        out_specs=[
            pl.BlockSpec((g, 1), lambda i: (0, 0)),
            pl.BlockSpec((g, d), lambda i: (0, 0)),
            pl.BlockSpec((g, d, d), lambda i: (0, 0, 0)),
        ],
        out_shape=[
            jax.ShapeDtypeStruct((g, 1), jnp.float32),
            jax.ShapeDtypeStruct((g, d), jnp.float32),
            jax.ShapeDtypeStruct((g, d, d), jnp.float32),
        ],
        compiler_params=pltpu.CompilerParams(
            dimension_semantics=("arbitrary",)),
    )(z2, X, r)

    return ns.reshape(g), mus, covs.reshape(g, d, d)


# speedup vs baseline: 1.2717x; 1.0366x over previous
import jax
import jax.numpy as jnp
from jax.experimental import pallas as pl
from jax.experimental.pallas import tpu as pltpu

K = 32
EPS = 1e-6
BLK = 1024


def _stats_kernel(z_ref, x_ref, r_ref, ns_ref, mus_ref, covs_ref,
                  *, nblocks, s, d):
    step = pl.program_id(0)
    g = K * s

    @pl.when(step == 0)
    def _init():
        ns_ref[...] = jnp.zeros_like(ns_ref)
        mus_ref[...] = jnp.zeros_like(mus_ref)
        covs_ref[...] = jnp.zeros_like(covs_ref)

    x = x_ref[...]
    r = r_ref[...]
    zc = z_ref[...]
    zmin = zc[0, 0]
    zmax = zc[BLK - 1, 0]

    lane = jax.lax.broadcasted_iota(jnp.int32, (BLK, g), 1)
    rcols = jnp.broadcast_to(r[:, 0:1], (BLK, g))
    for j in range(1, s):
        rcols = jnp.where(lane % s == j,
                          jnp.broadcast_to(r[:, j:j + 1], (BLK, g)), rcols)
    w_dense = jnp.where(zc == lane // s, rcols, 0.0)
    xa = jnp.concatenate(
        [x, jnp.ones((BLK, 1), jnp.float32)], axis=1)
    y2 = jax.lax.dot_general(
        w_dense, xa, (((0,), (0,)), ((), ())),
        preferred_element_type=jnp.float32)
    mus_ref[...] += y2[:, 0:d]
    ns_ref[...] += y2[:, d:d + 1]

    a = jnp.concatenate([x * r[:, j:j + 1] for j in range(s)],
                        axis=1)

    def seg_body(i, carry):
        for u in range(2):
            k = zmin + i * 2 + u
            xm = jnp.where(zc == k, x, 0.0)
            y = jax.lax.dot_general(
                a, xm, (((0,), (0,)), ((), ())),
                preferred_element_type=jnp.float32)
            kc = jnp.minimum(k, K - 1)
            for j in range(s):
                covs_ref[pl.ds(kc * s + j, 1)] += (
                    y[j * d:(j + 1) * d, :][None])
        return carry

    jax.lax.fori_loop(0, (zmax - zmin + 2) // 2, seg_body, 0)

    @pl.when(step == nblocks - 1)
    def _epilogue():
        denom = ns_ref[...] + EPS
        mus = mus_ref[...] / denom
        mus_ref[...] = mus
        covs_ref[...] = (covs_ref[...] / denom[:, :, None]
                         - mus[:, :, None] * mus[:, None, :])


def kernel(X, z, r):
    n, d = X.shape
    s = r.shape[1]
    g = K * s
    nblocks = n // BLK
    z2 = z.astype(jnp.int32).reshape(n, 1)

    ns, mus, covs = pl.pallas_call(
        lambda *refs: _stats_kernel(*refs, nblocks=nblocks, s=s, d=d),
        grid=(nblocks,),
        in_specs=[
            pl.BlockSpec((BLK, 1), lambda i: (i, 0)),
            pl.BlockSpec((BLK, d), lambda i: (i, 0)),
            pl.BlockSpec((BLK, s), lambda i: (i, 0)),
        ],
        out_specs=[
            pl.BlockSpec((g, 1), lambda i: (0, 0)),
            pl.BlockSpec((g, d), lambda i: (0, 0)),
            pl.BlockSpec((g, d, d), lambda i: (0, 0, 0)),
        ],
        out_shape=[
            jax.ShapeDtypeStruct((g, 1), jnp.float32),
            jax.ShapeDtypeStruct((g, d), jnp.float32),
            jax.ShapeDtypeStruct((g, d, d), jnp.float32),
        ],
        compiler_params=pltpu.CompilerParams(
            dimension_semantics=("arbitrary",)),
    )(z2, X, r)

    return ns.reshape(g), mus, covs.reshape(g, d, d)
